# baseline (speedup 1.0000x reference)
import functools

import jax
import jax.numpy as jnp
from jax import lax
from jax.experimental import pallas as pl
from jax.experimental.pallas import tpu as pltpu
from jax.experimental.pallas import tpu_sc as plsc

_N = 10000
_E = 320000
_D = 128
_NC = 2
_NS = 16
_NW = _NC * _NS
_EPW = _E // _NW
_CH = 80
_NCH = _EPW // _CH
_NP = 10240
_RPT = _NP // _NS
_DEGW = 16

_mesh = plsc.VectorSubcoreMesh(core_axis_name="c", subcore_axis_name="s")


@functools.partial(
    pl.kernel,
    out_type=jax.ShapeDtypeStruct((_NC, _NP), jnp.float32),
    mesh=_mesh,
    scratch_types=[
        pltpu.VMEM((_NCH, _CH), jnp.int32),
        pltpu.VMEM((_CH,), jnp.float32),
        pltpu.VMEM((_RPT,), jnp.float32),
        pltpu.VMEM_SHARED((_NP,), jnp.float32),
    ],
)
def _deg_kernel(dst_hbm, out_hbm, didx_v, ones_v, zbuf_v, deg_sh):
    c = lax.axis_index("c")
    s = lax.axis_index("s")
    wid = c * _NS + s
    base = pl.multiple_of(s * _RPT, 8)
    pltpu.sync_copy(dst_hbm.at[wid], didx_v)

    @pl.loop(0, _CH // 16)
    def _(i):
        ones_v[pl.ds(i * 16, 16)] = jnp.full((16,), 1.0, jnp.float32)

    @pl.loop(0, _RPT // 16)
    def _(i):
        zbuf_v[pl.ds(i * 16, 16)] = jnp.zeros((16,), jnp.float32)

    pltpu.sync_copy(zbuf_v, deg_sh.at[pl.ds(base, _RPT)])
    plsc.subcore_barrier()

    @pl.loop(0, _NCH)
    def _(j):
        pltpu.sync_copy(ones_v, deg_sh.at[didx_v.at[j]], add=True)

    plsc.subcore_barrier()
    pltpu.sync_copy(deg_sh.at[pl.ds(base, _RPT)],
                    out_hbm.at[c].at[pl.ds(base, _RPT)])


@functools.partial(
    pl.kernel,
    out_type=jax.ShapeDtypeStruct((_NC, _NP, _D), jnp.float32),
    mesh=_mesh,
    scratch_types=[
        pltpu.VMEM((_EPW,), jnp.int32),
        pltpu.VMEM((_NCH, _CH), jnp.int32),
        pltpu.VMEM((_CH, _D), jnp.float32),
        pltpu.VMEM((_CH, _D), jnp.float32),
        pltpu.VMEM_SHARED((_NP, _D), jnp.float32),
        pltpu.SemaphoreType.DMA,
        pltpu.SemaphoreType.DMA,
        pltpu.SemaphoreType.DMA,
        pltpu.SemaphoreType.DMA,
    ],
)
def _conv_kernel(g_hbm, src_hbm, dst_hbm, zeros_hbm, acc_hbm,
                 sidx_v, didx_v, rows0_v, rows1_v, acc_sh,
                 sem0, sem1, ssem0, ssem1):
    c = lax.axis_index("c")
    s = lax.axis_index("s")
    wid = c * _NS + s
    base = pl.multiple_of(s * _RPT, 8)
    pltpu.sync_copy(src_hbm.at[wid], sidx_v)
    pltpu.sync_copy(dst_hbm.at[wid], didx_v)
    pltpu.sync_copy(zeros_hbm.at[pl.ds(base, _RPT)],
                    acc_sh.at[pl.ds(base, _RPT)])
    plsc.subcore_barrier()

    def _src(j):
        return sidx_v.at[pl.ds(pl.multiple_of(j * _CH, 8), _CH)]

    pltpu.async_copy(g_hbm.at[_src(0)], rows0_v, sem0)

    @pl.loop(0, _NCH // 2)
    def _(p):
        j0 = p * 2
        pltpu.make_async_copy(g_hbm.at[_src(j0)], rows0_v, sem0).wait()

        @pl.when(j0 > 0)
        def _():
            pltpu.make_async_copy(rows1_v, acc_sh.at[didx_v.at[j0]],
                                  ssem1).wait()

        pltpu.async_copy(g_hbm.at[_src(j0 + 1)], rows1_v, sem1)
        pltpu.async_copy(rows0_v, acc_sh.at[didx_v.at[j0]], ssem0, add=True)
        pltpu.make_async_copy(g_hbm.at[_src(j0 + 1)], rows1_v, sem1).wait()
        pltpu.make_async_copy(rows0_v, acc_sh.at[didx_v.at[j0]],
                              ssem0).wait()
        pltpu.async_copy(g_hbm.at[_src(j0 + 2)], rows0_v, sem0)
        pltpu.async_copy(rows1_v, acc_sh.at[didx_v.at[j0 + 1]], ssem1,
                         add=True)

    pltpu.make_async_copy(g_hbm.at[_src(_NCH - 1)], rows0_v, sem0).wait()
    pltpu.make_async_copy(rows1_v, acc_sh.at[didx_v.at[0]], ssem1).wait()
    pltpu.sync_copy(rows0_v, acc_sh.at[didx_v.at[_NCH - 1]], add=True)

    plsc.subcore_barrier()
    pltpu.sync_copy(acc_sh.at[pl.ds(base, _RPT)],
                    acc_hbm.at[c].at[pl.ds(base, _RPT)])


def _tc1(deg_ref, x_ref, w_ref, g_ref, dinv_ref):
    deg = (deg_ref[0, :_N] + deg_ref[1, :_N] + 1.0).reshape(_N, 1)
    dinv = lax.rsqrt(deg)
    h = jnp.dot(x_ref[...], w_ref[...], preferred_element_type=jnp.float32)
    g_ref[...] = h * dinv
    dinv_ref[...] = dinv


def _tc2(acc_ref, g1_ref, dinv_ref, b1_ref, w2_ref, g2_ref):
    dinv = dinv_ref[...]
    h1 = jnp.maximum((acc_ref[0, :_N] + acc_ref[1, :_N] + g1_ref[...]) * dinv
                     + b1_ref[...], 0.0)
    g2_ref[...] = jnp.dot(h1, w2_ref[...],
                          preferred_element_type=jnp.float32) * dinv


def _tc3(acc_ref, g2_ref, dinv_ref, b2_ref, aw_ref, att_ref, probs_ref):
    dinv = dinv_ref[...]
    h2 = jnp.maximum((acc_ref[0, :_N] + acc_ref[1, :_N] + g2_ref[...]) * dinv
                     + b2_ref[...], 0.0)
    sc = jnp.dot(h2, aw_ref[...], preferred_element_type=jnp.float32)
    e = jnp.exp(sc - jnp.max(sc))
    p = e / jnp.sum(e)
    att_ref[...] = h2 * p
    probs_ref[...] = p


_tc1_call = pl.pallas_call(
    _tc1,
    out_shape=(jax.ShapeDtypeStruct((_N, _D), jnp.float32),
               jax.ShapeDtypeStruct((_N, 1), jnp.float32)),
)
_tc2_call = pl.pallas_call(
    _tc2,
    out_shape=jax.ShapeDtypeStruct((_N, _D), jnp.float32),
)
_tc3_call = pl.pallas_call(
    _tc3,
    out_shape=(jax.ShapeDtypeStruct((_N, _D), jnp.float32),
               jax.ShapeDtypeStruct((_N, 1), jnp.float32)),
)


def kernel(x, edge_index, W1, b1, W2, b2, att_w):
    src2 = edge_index[0].reshape(_NW, _EPW)
    dst3 = edge_index[1].reshape(_NW, _NCH, _CH)
    z128 = jnp.zeros((_NP, _D), jnp.float32)
    b1r = b1.reshape(1, _D)
    b2r = b2.reshape(1, _D)

    deg_parts = _deg_kernel(dst3)
    g1, dinv = _tc1_call(deg_parts, x, W1)
    acc1 = _conv_kernel(g1, src2, dst3, z128)
    g2 = _tc2_call(acc1, g1, dinv, b1r, W2)
    acc2 = _conv_kernel(g2, src2, dst3, z128)
    att, probs = _tc3_call(acc2, g2, dinv, b2r, att_w)
    return att, probs.reshape(_N)


# speedup vs baseline: 1.0210x; 1.0210x over previous
import functools

import jax
import jax.numpy as jnp
from jax import lax
from jax.experimental import pallas as pl
from jax.experimental.pallas import tpu as pltpu
from jax.experimental.pallas import tpu_sc as plsc

_N = 10000
_E = 320000
_D = 128
_NC = 2
_NS = 16
_NW = _NC * _NS
_EPW = _E // _NW
_CH = 80
_NCH = _EPW // _CH
_NP = 10240
_RPT = _NP // _NS
_DEGW = 16

_mesh = plsc.VectorSubcoreMesh(core_axis_name="c", subcore_axis_name="s")


@functools.partial(
    pl.kernel,
    out_type=jax.ShapeDtypeStruct((_NC, _NP), jnp.float32),
    mesh=_mesh,
    scratch_types=[
        pltpu.VMEM((_NCH, _CH), jnp.int32),
        pltpu.VMEM((_CH,), jnp.float32),
        pltpu.VMEM((_RPT,), jnp.float32),
        pltpu.VMEM_SHARED((_NP,), jnp.float32),
    ],
)
def _deg_kernel(dst_hbm, out_hbm, didx_v, ones_v, zbuf_v, deg_sh):
    c = lax.axis_index("c")
    s = lax.axis_index("s")
    wid = c * _NS + s
    base = pl.multiple_of(s * _RPT, 8)
    pltpu.sync_copy(dst_hbm.at[wid], didx_v)

    @pl.loop(0, _CH // 16)
    def _(i):
        ones_v[pl.ds(i * 16, 16)] = jnp.full((16,), 1.0, jnp.float32)

    @pl.loop(0, _RPT // 16)
    def _(i):
        zbuf_v[pl.ds(i * 16, 16)] = jnp.zeros((16,), jnp.float32)

    pltpu.sync_copy(zbuf_v, deg_sh.at[pl.ds(base, _RPT)])
    plsc.subcore_barrier()

    @pl.loop(0, _NCH)
    def _(j):
        pltpu.sync_copy(ones_v, deg_sh.at[didx_v.at[j]], add=True)

    plsc.subcore_barrier()
    pltpu.sync_copy(deg_sh.at[pl.ds(base, _RPT)],
                    out_hbm.at[c].at[pl.ds(base, _RPT)])


@functools.partial(
    pl.kernel,
    out_type=jax.ShapeDtypeStruct((_NC, _NP, _D), jnp.float32),
    mesh=_mesh,
    scratch_types=[
        pltpu.VMEM((_EPW,), jnp.int32),
        pltpu.VMEM((_NCH, _CH), jnp.int32),
        pltpu.VMEM((_CH, _D), jnp.float32),
        pltpu.VMEM((_CH, _D), jnp.float32),
        pltpu.VMEM_SHARED((_NP, _D), jnp.float32),
        pltpu.SemaphoreType.DMA,
        pltpu.SemaphoreType.DMA,
    ],
)
def _conv_kernel(g_hbm, src_hbm, dst_hbm, acc_hbm,
                 sidx_v, didx_v, rows0_v, rows1_v, acc_sh, sem0, sem1):
    c = lax.axis_index("c")
    s = lax.axis_index("s")
    wid = c * _NS + s
    base = pl.multiple_of(s * _RPT, 8)
    pltpu.sync_copy(src_hbm.at[wid], sidx_v)
    pltpu.sync_copy(dst_hbm.at[wid], didx_v)

    @pl.loop(0, _CH)
    def _(i):
        for k in range(_D // 16):
            rows0_v[i, pl.ds(k * 16, 16)] = jnp.zeros((16,), jnp.float32)

    @pl.loop(0, _RPT // _CH)
    def _(i):
        off = pl.multiple_of(base + i * _CH, 8)
        pltpu.sync_copy(rows0_v, acc_sh.at[pl.ds(off, _CH)])

    plsc.subcore_barrier()

    def _src(j):
        return sidx_v.at[pl.ds(pl.multiple_of(j * _CH, 8), _CH)]

    pltpu.async_copy(g_hbm.at[_src(0)], rows0_v, sem0)

    @pl.loop(0, _NCH // 2)
    def _(p):
        j0 = p * 2
        pltpu.make_async_copy(g_hbm.at[_src(j0)], rows0_v, sem0).wait()
        pltpu.async_copy(g_hbm.at[_src(j0 + 1)], rows1_v, sem1)
        pltpu.sync_copy(rows0_v, acc_sh.at[didx_v.at[j0]], add=True)
        pltpu.make_async_copy(g_hbm.at[_src(j0 + 1)], rows1_v, sem1).wait()
        pltpu.async_copy(g_hbm.at[_src(j0 + 2)], rows0_v, sem0)
        pltpu.sync_copy(rows1_v, acc_sh.at[didx_v.at[j0 + 1]], add=True)

    pltpu.make_async_copy(g_hbm.at[_src(_NCH - 1)], rows0_v, sem0).wait()
    pltpu.sync_copy(rows0_v, acc_sh.at[didx_v.at[_NCH - 1]], add=True)

    plsc.subcore_barrier()
    pltpu.sync_copy(acc_sh.at[pl.ds(base, _RPT)],
                    acc_hbm.at[c].at[pl.ds(base, _RPT)])


def _tc1(deg_ref, x_ref, w_ref, g_ref, dinv_ref):
    deg = (deg_ref[0, :_N] + deg_ref[1, :_N] + 1.0).reshape(_N, 1)
    dinv = lax.rsqrt(deg)
    h = jnp.dot(x_ref[...], w_ref[...], preferred_element_type=jnp.float32)
    g_ref[...] = h * dinv
    dinv_ref[...] = dinv


def _tc2(acc_ref, g1_ref, dinv_ref, b1_ref, w2_ref, g2_ref):
    dinv = dinv_ref[...]
    h1 = jnp.maximum((acc_ref[0, :_N] + acc_ref[1, :_N] + g1_ref[...]) * dinv
                     + b1_ref[...], 0.0)
    g2_ref[...] = jnp.dot(h1, w2_ref[...],
                          preferred_element_type=jnp.float32) * dinv


def _tc3(acc_ref, g2_ref, dinv_ref, b2_ref, aw_ref, att_ref, probs_ref):
    dinv = dinv_ref[...]
    h2 = jnp.maximum((acc_ref[0, :_N] + acc_ref[1, :_N] + g2_ref[...]) * dinv
                     + b2_ref[...], 0.0)
    sc = jnp.dot(h2, aw_ref[...], preferred_element_type=jnp.float32)
    e = jnp.exp(sc - jnp.max(sc))
    p = e / jnp.sum(e)
    att_ref[...] = h2 * p
    probs_ref[...] = p


_tc1_call = pl.pallas_call(
    _tc1,
    out_shape=(jax.ShapeDtypeStruct((_N, _D), jnp.float32),
               jax.ShapeDtypeStruct((_N, 1), jnp.float32)),
)
_tc2_call = pl.pallas_call(
    _tc2,
    out_shape=jax.ShapeDtypeStruct((_N, _D), jnp.float32),
)
_tc3_call = pl.pallas_call(
    _tc3,
    out_shape=(jax.ShapeDtypeStruct((_N, _D), jnp.float32),
               jax.ShapeDtypeStruct((_N, 1), jnp.float32)),
)


def kernel(x, edge_index, W1, b1, W2, b2, att_w):
    src2 = edge_index[0].reshape(_NW, _EPW)
    dst3 = edge_index[1].reshape(_NW, _NCH, _CH)
    b1r = b1.reshape(1, _D)
    b2r = b2.reshape(1, _D)

    deg_parts = _deg_kernel(dst3)
    g1, dinv = _tc1_call(deg_parts, x, W1)
    acc1 = _conv_kernel(g1, src2, dst3)
    g2 = _tc2_call(acc1, g1, dinv, b1r, W2)
    acc2 = _conv_kernel(g2, src2, dst3)
    att, probs = _tc3_call(acc2, g2, dinv, b2r, att_w)
    return att, probs.reshape(_N)


# speedup vs baseline: 1.0398x; 1.0183x over previous
import functools

import jax
import jax.numpy as jnp
from jax import lax
from jax.experimental import pallas as pl
from jax.experimental.pallas import tpu as pltpu
from jax.experimental.pallas import tpu_sc as plsc

_N = 10000
_E = 320000
_D = 128
_NC = 2
_NS = 16
_NW = _NC * _NS
_EPW = _E // _NW
_CH = 80
_NCH = _EPW // _CH
_NP = 10240
_RPT = _NP // _NS
_DEGW = 16

_mesh = plsc.VectorSubcoreMesh(core_axis_name="c", subcore_axis_name="s")


@functools.partial(
    pl.kernel,
    out_type=jax.ShapeDtypeStruct((_NC, _NP), jnp.float32),
    mesh=_mesh,
    scratch_types=[
        pltpu.VMEM((_NCH, _CH), jnp.int32),
        pltpu.VMEM((_CH,), jnp.float32),
        pltpu.VMEM((_RPT,), jnp.float32),
        pltpu.VMEM_SHARED((_NP,), jnp.float32),
        pltpu.SemaphoreType.DMA,
    ],
)
def _deg_kernel(dst_hbm, out_hbm, didx_v, ones_v, zbuf_v, deg_sh, dsem):
    c = lax.axis_index("c")
    s = lax.axis_index("s")
    wid = c * _NS + s
    base = pl.multiple_of(s * _RPT, 8)
    pltpu.sync_copy(dst_hbm.at[wid], didx_v)

    @pl.loop(0, _CH // 16)
    def _(i):
        ones_v[pl.ds(i * 16, 16)] = jnp.full((16,), 1.0, jnp.float32)

    @pl.loop(0, _RPT // 16)
    def _(i):
        zbuf_v[pl.ds(i * 16, 16)] = jnp.zeros((16,), jnp.float32)

    pltpu.sync_copy(zbuf_v, deg_sh.at[pl.ds(base, _RPT)])
    plsc.subcore_barrier()

    @pl.loop(0, _NCH // 5)
    def _(b):
        j0 = b * 5
        for k in range(5):
            pltpu.async_copy(ones_v, deg_sh.at[didx_v.at[j0 + k]], dsem,
                             add=True)
        for k in range(5):
            pltpu.make_async_copy(ones_v, deg_sh.at[didx_v.at[j0]],
                                  dsem).wait()

    plsc.subcore_barrier()
    pltpu.sync_copy(deg_sh.at[pl.ds(base, _RPT)],
                    out_hbm.at[c].at[pl.ds(base, _RPT)])


@functools.partial(
    pl.kernel,
    out_type=jax.ShapeDtypeStruct((_NC, _NP, _D), jnp.float32),
    mesh=_mesh,
    scratch_types=[
        pltpu.VMEM((_EPW,), jnp.int32),
        pltpu.VMEM((_NCH, _CH), jnp.int32),
        pltpu.VMEM((_CH, _D), jnp.float32),
        pltpu.VMEM((_CH, _D), jnp.float32),
        pltpu.VMEM_SHARED((_NP, _D), jnp.float32),
        pltpu.SemaphoreType.DMA,
        pltpu.SemaphoreType.DMA,
    ],
)
def _conv_kernel(g_hbm, src_hbm, dst_hbm, acc_hbm,
                 sidx_v, didx_v, rows0_v, rows1_v, acc_sh, sem0, sem1):
    c = lax.axis_index("c")
    s = lax.axis_index("s")
    wid = c * _NS + s
    base = pl.multiple_of(s * _RPT, 8)
    pltpu.sync_copy(src_hbm.at[wid], sidx_v)
    pltpu.sync_copy(dst_hbm.at[wid], didx_v)

    @pl.loop(0, _CH)
    def _(i):
        for k in range(_D // 16):
            rows0_v[i, pl.ds(k * 16, 16)] = jnp.zeros((16,), jnp.float32)

    @pl.loop(0, _RPT // _CH)
    def _(i):
        off = pl.multiple_of(base + i * _CH, 8)
        pltpu.sync_copy(rows0_v, acc_sh.at[pl.ds(off, _CH)])

    plsc.subcore_barrier()

    def _src(j):
        return sidx_v.at[pl.ds(pl.multiple_of(j * _CH, 8), _CH)]

    pltpu.async_copy(g_hbm.at[_src(0)], rows0_v, sem0)

    @pl.loop(0, _NCH // 2)
    def _(p):
        j0 = p * 2
        pltpu.make_async_copy(g_hbm.at[_src(j0)], rows0_v, sem0).wait()
        pltpu.async_copy(g_hbm.at[_src(j0 + 1)], rows1_v, sem1)
        pltpu.sync_copy(rows0_v, acc_sh.at[didx_v.at[j0]], add=True)
        pltpu.make_async_copy(g_hbm.at[_src(j0 + 1)], rows1_v, sem1).wait()
        pltpu.async_copy(g_hbm.at[_src(j0 + 2)], rows0_v, sem0)
        pltpu.sync_copy(rows1_v, acc_sh.at[didx_v.at[j0 + 1]], add=True)

    pltpu.make_async_copy(g_hbm.at[_src(_NCH - 1)], rows0_v, sem0).wait()
    pltpu.sync_copy(rows0_v, acc_sh.at[didx_v.at[_NCH - 1]], add=True)

    plsc.subcore_barrier()
    pltpu.sync_copy(acc_sh.at[pl.ds(base, _RPT)],
                    acc_hbm.at[c].at[pl.ds(base, _RPT)])


def _tc1(deg_ref, x_ref, w_ref, g_ref, dinv_ref):
    deg = (deg_ref[0, :_N] + deg_ref[1, :_N] + 1.0).reshape(_N, 1)
    dinv = lax.rsqrt(deg)
    h = jnp.dot(x_ref[...], w_ref[...], preferred_element_type=jnp.float32)
    g_ref[...] = h * dinv
    dinv_ref[...] = dinv


def _tc2(acc_ref, g1_ref, dinv_ref, b1_ref, w2_ref, g2_ref):
    dinv = dinv_ref[...]
    h1 = jnp.maximum((acc_ref[0, :_N] + acc_ref[1, :_N] + g1_ref[...]) * dinv
                     + b1_ref[...], 0.0)
    g2_ref[...] = jnp.dot(h1, w2_ref[...],
                          preferred_element_type=jnp.float32) * dinv


def _tc3(acc_ref, g2_ref, dinv_ref, b2_ref, aw_ref, att_ref, probs_ref):
    dinv = dinv_ref[...]
    h2 = jnp.maximum((acc_ref[0, :_N] + acc_ref[1, :_N] + g2_ref[...]) * dinv
                     + b2_ref[...], 0.0)
    sc = jnp.dot(h2, aw_ref[...], preferred_element_type=jnp.float32)
    e = jnp.exp(sc - jnp.max(sc))
    p = e / jnp.sum(e)
    att_ref[...] = h2 * p
    probs_ref[...] = p


_tc1_call = pl.pallas_call(
    _tc1,
    out_shape=(jax.ShapeDtypeStruct((_N, _D), jnp.float32),
               jax.ShapeDtypeStruct((_N, 1), jnp.float32)),
)
_tc2_call = pl.pallas_call(
    _tc2,
    out_shape=jax.ShapeDtypeStruct((_N, _D), jnp.float32),
)
_tc3_call = pl.pallas_call(
    _tc3,
    out_shape=(jax.ShapeDtypeStruct((_N, _D), jnp.float32),
               jax.ShapeDtypeStruct((_N, 1), jnp.float32)),
)


def kernel(x, edge_index, W1, b1, W2, b2, att_w):
    src2 = edge_index[0].reshape(_NW, _EPW)
    dst3 = edge_index[1].reshape(_NW, _NCH, _CH)
    b1r = b1.reshape(1, _D)
    b2r = b2.reshape(1, _D)

    deg_parts = _deg_kernel(dst3)
    g1, dinv = _tc1_call(deg_parts, x, W1)
    acc1 = _conv_kernel(g1, src2, dst3)
    g2 = _tc2_call(acc1, g1, dinv, b1r, W2)
    acc2 = _conv_kernel(g2, src2, dst3)
    att, probs = _tc3_call(acc2, g2, dinv, b2r, att_w)
    return att, probs.reshape(_N)
